# baseline (speedup 1.0000x reference)
import jax
import jax.numpy as jnp
from jax import lax
from jax.experimental import pallas as pl
from jax.experimental.pallas import tpu as pltpu
from jax.experimental.pallas import tpu_sc as plsc

VOCAB = 100000
DIM = 128
NB = 16384
KTOP = 8
LANES = 16
NCORE = 2
NSUB = 16
NW = NCORE * NSUB
PPW = NB // NW
CHA = 128
NREG = DIM // LANES
NSLICE = 12
SLICE_R = 8400
PPT = NB // NSUB
MAXM = 2 * PPT
CHB = 16
CHL = 128
MAXCH = MAXM // CHL + 1
UZERO = 2 * NB


def _rsqrt16(x):
  xi = plsc.bitcast(x, jnp.int32)
  yi = jnp.int32(0x5F3759DF) - lax.shift_right_logical(xi, 1)
  y = plsc.bitcast(yi, jnp.float32)
  for _ in range(3):
    y = y * (1.5 - 0.5 * x * y * y)
  return y


def _sort16(v):
  return lax.sort(v, dimension=0, num_keys=1)


def _merge16(a, b):
  return _sort16(jnp.maximum(a, lax.rev(b, (0,))))


def _top16(vals):
  t = [_sort16(v) for v in vals]
  while len(t) > 1:
    t = [_merge16(t[2 * i], t[2 * i + 1]) for i in range(len(t) // 2)]
  return t[0]


def _phase_a_body(table, ida, idb, alphav, u_out,
                  ida_v, idb_v, al_v, ea, eb, ua, ub, zb,
                  sem_a, sem_b, sem_sa, sem_sb):
  cid = lax.axis_index("c")
  sid = lax.axis_index("s")
  wid = sid * NCORE + cid
  base = wid * PPW
  cpa = pltpu.async_copy(ida.at[pl.ds(base, PPW)], ida_v, sem_a)
  cpb = pltpu.async_copy(idb.at[pl.ds(base, PPW)], idb_v, sem_b)
  pltpu.sync_copy(alphav, al_v)
  alpha16 = al_v[...]

  z = jnp.zeros((LANES,), jnp.float32)
  for i in range(CHB):
    for j in range(NREG):
      zb[i, pl.ds(j * LANES, LANES)] = z

  @pl.when(wid == 0)
  def _():
    pltpu.sync_copy(zb, u_out.at[pl.ds(UZERO, CHB)])

  cpa.wait()
  cpb.wait()

  iot = lax.iota(jnp.int32, LANES)
  topmask = iot >= (LANES - KTOP)
  inf16 = jnp.full((LANES,), jnp.inf, jnp.float32)

  prev_stores = []
  for ci in range(PPW // CHA):
    ga = pltpu.async_copy(table.at[ida_v.at[pl.ds(ci * CHA, CHA)]], ea, sem_a)
    gb = pltpu.async_copy(table.at[idb_v.at[pl.ds(ci * CHA, CHA)]], eb, sem_b)
    ga.wait()
    gb.wait()
    for d in prev_stores:
      d.wait()

    def pair_body(p, carry):
      a = [ea[p, pl.ds(j * LANES, LANES)] for j in range(NREG)]
      b = [eb[p, pl.ds(j * LANES, LANES)] for j in range(NREG)]
      al = [jnp.abs(a[j] * b[j]) for j in range(NREG)]
      d = [a[j] - b[j] for j in range(NREG)]
      ss = d[0] * d[0]
      for j in range(1, NREG):
        ss = ss + d[j] * d[j]
      tot = jnp.sum(ss)
      x16 = jnp.full((LANES,), tot, jnp.float32)
      y16 = _rsqrt16(x16)
      t16 = _top16(al)
      thr = jnp.min(jnp.where(topmask, t16, inf16))
      thr16 = jnp.full((LANES,), thr, jnp.float32)
      s16 = alpha16 * y16
      for j in range(NREG):
        u = jnp.where(al[j] >= thr16, s16 * d[j], 0.0)
        ua[p, pl.ds(j * LANES, LANES)] = u
        ub[p, pl.ds(j * LANES, LANES)] = -u
      return carry

    lax.fori_loop(0, CHA, pair_body, 0, unroll=2)
    sa = pltpu.async_copy(ua, u_out.at[pl.ds(base + ci * CHA, CHA)], sem_sa)
    sb = pltpu.async_copy(ub, u_out.at[pl.ds(NB + base + ci * CHA, CHA)],
                          sem_sb)
    prev_stores = [sa, sb]
  for d in prev_stores:
    d.wait()


def _phase_b_body(table, ida, idb, u_in, out,
                  ida_v, idb_v, rows_l, items_l, lrows_l, zb, ubuf, tbuf,
                  sem1, sem2, sh_delta):
  cid = lax.axis_index("c")
  sid = lax.axis_index("s")
  tbase = sid * PPT
  cpa = pltpu.async_copy(ida.at[pl.ds(tbase, PPT)], ida_v, sem1)
  cpb = pltpu.async_copy(idb.at[pl.ds(tbase, PPT)], idb_v, sem2)

  z = jnp.zeros((LANES,), jnp.float32)

  def zb_body(i, carry):
    for j in range(NREG):
      zb[i, pl.ds(j * LANES, LANES)] = z
    return carry
  lax.fori_loop(0, CHL, zb_body, 0)

  cpa.wait()
  cpb.wait()
  iot = lax.iota(jnp.int32, LANES)

  for sl in range(NSLICE // NCORE):
    s = sl * NCORE + cid
    lo = s * SLICE_R
    lo16 = jnp.full((LANES,), lo, jnp.int32)
    uz16 = jnp.full((LANES,), UZERO, jnp.int32)

    def fill_body(q, carry):
      for j in range(NREG):
        rows_l[q, pl.ds(j * LANES, LANES)] = lo16
        items_l[q, pl.ds(j * LANES, LANES)] = uz16
      return carry
    lax.fori_loop(0, MAXCH, fill_body, 0)

    def make_scan(idv, item_off):
      def ch_body(ch, cnt):
        r = idv[pl.ds(ch * LANES, LANES)]
        m = (r >= lo16) & (r < lo16 + SLICE_R)
        pos = cnt + plsc.cumsum(m.astype(jnp.int32)) - 1
        i0 = lax.shift_right_logical(pos, 7)
        i1 = pos & (CHL - 1)
        plsc.store_scatter(rows_l, [i0, i1], r, mask=m)
        it = item_off + ch * LANES + iot
        plsc.store_scatter(items_l, [i0, i1], it, mask=m)
        return cnt + plsc.all_reduce_population_count(m)
      return ch_body

    cnt = jnp.zeros((LANES,), jnp.int32)
    cnt = lax.fori_loop(0, PPT // LANES, make_scan(ida_v, tbase), cnt)
    cnt = lax.fori_loop(0, PPT // LANES, make_scan(idb_v, NB + tbase), cnt)
    nch = lax.shift_right_logical(jnp.max(cnt) + (CHL - 1), 7)

    def lr_body(q, carry):
      for j in range(NREG):
        lrows_l[q, pl.ds(j * LANES, LANES)] = (
            rows_l[q, pl.ds(j * LANES, LANES)] - lo16)
      return carry
    lax.fori_loop(0, MAXCH, lr_body, 0)

    @pl.when(sid == 0)
    def _():
      pltpu.sync_copy(zb.at[pl.ds(0, CHB)], sh_delta.at[pl.ds(0, CHB)])

    def zch(i, carry):
      pltpu.sync_copy(zb, sh_delta.at[lrows_l.at[i]])
      return carry
    lax.fori_loop(0, nch, zch, 0)
    plsc.subcore_barrier()

    def ach(i, carry):
      pltpu.sync_copy(u_in.at[items_l.at[i]], ubuf)
      pltpu.sync_copy(ubuf, sh_delta.at[lrows_l.at[i]], add=True)
      return carry
    lax.fori_loop(0, nch, ach, 0)
    plsc.subcore_barrier()

    def pch(i, carry):
      c1 = pltpu.async_copy(table.at[rows_l.at[i]], tbuf, sem1)
      c2 = pltpu.async_copy(sh_delta.at[lrows_l.at[i]], ubuf, sem2)
      c1.wait()
      c2.wait()

      def add_body(rr, cc):
        for j in range(NREG):
          tbuf[rr, pl.ds(j * LANES, LANES)] = (
              tbuf[rr, pl.ds(j * LANES, LANES)]
              + ubuf[rr, pl.ds(j * LANES, LANES)])
        return cc
      lax.fori_loop(0, CHL, add_body, 0)
      pltpu.sync_copy(tbuf, out.at[rows_l.at[i]])
      return carry
    lax.fori_loop(0, nch, pch, 0)
    plsc.subcore_barrier()


def _make_kernels():
  mesh = plsc.VectorSubcoreMesh(core_axis_name="c", subcore_axis_name="s",
                                num_cores=NCORE, num_subcores=NSUB)
  params = pltpu.CompilerParams(needs_layout_passes=False)
  phase_a = pl.kernel(
      _phase_a_body,
      out_type=jax.ShapeDtypeStruct((2 * NB + CHB, DIM), jnp.float32),
      mesh=mesh,
      compiler_params=params,
      scratch_types=[
          pltpu.VMEM((PPW,), jnp.int32),
          pltpu.VMEM((PPW,), jnp.int32),
          pltpu.VMEM((LANES,), jnp.float32),
          pltpu.VMEM((CHA, DIM), jnp.float32),
          pltpu.VMEM((CHA, DIM), jnp.float32),
          pltpu.VMEM((CHA, DIM), jnp.float32),
          pltpu.VMEM((CHA, DIM), jnp.float32),
          pltpu.VMEM((CHB, DIM), jnp.float32),
          pltpu.SemaphoreType.DMA,
          pltpu.SemaphoreType.DMA,
          pltpu.SemaphoreType.DMA,
          pltpu.SemaphoreType.DMA,
      ],
  )
  phase_b = pl.kernel(
      _phase_b_body,
      out_type=(),
      mesh=mesh,
      compiler_params=params,
      scratch_types=[
          pltpu.VMEM((PPT,), jnp.int32),
          pltpu.VMEM((PPT,), jnp.int32),
          pltpu.VMEM((MAXCH, CHL), jnp.int32),
          pltpu.VMEM((MAXCH, CHL), jnp.int32),
          pltpu.VMEM((MAXCH, CHL), jnp.int32),
          pltpu.VMEM((CHL, DIM), jnp.float32),
          pltpu.VMEM((CHL, DIM), jnp.float32),
          pltpu.VMEM((CHL, DIM), jnp.float32),
          pltpu.SemaphoreType.DMA,
          pltpu.SemaphoreType.DMA,
          pltpu.VMEM_SHARED((SLICE_R, DIM), jnp.float32),
      ],
  )
  return phase_a, phase_b


_PHASE_A, _PHASE_B = None, None


def kernel(table, token_ids_a, token_ids_b, alpha):
  global _PHASE_A, _PHASE_B
  if _PHASE_A is None:
    _PHASE_A, _PHASE_B = _make_kernels()
  alpha16 = jnp.broadcast_to(alpha.astype(jnp.float32), (LANES,))
  u = _PHASE_A(table, token_ids_a, token_ids_b, alpha16)
  out_ref = jax.new_ref(table)
  _PHASE_B(table, token_ids_a, token_ids_b, u, out_ref)
  return jax.freeze(out_ref)


# speedup vs baseline: 5.6706x; 5.6706x over previous
import jax
import jax.numpy as jnp
from jax import lax
from jax.experimental import pallas as pl
from jax.experimental.pallas import tpu as pltpu
from jax.experimental.pallas import tpu_sc as plsc

VOCAB = 100000
DIM = 128
NB = 16384
KTOP = 8
LANES = 16
NCORE = 2
NSUB = 16
NW = NCORE * NSUB
PPW = NB // NW
CHA = 128
NREG = DIM // LANES
NSLICE = 12
SLICE_R = 8400
PPT = NB // NSUB
MAXM = 2 * PPT
CHB = 16
CHL = 128
MAXCH = MAXM // CHL + 1
UZERO = 2 * NB


def _rsqrt16(x):
  xi = plsc.bitcast(x, jnp.int32)
  yi = jnp.int32(0x5F3759DF) - lax.shift_right_logical(xi, 1)
  y = plsc.bitcast(yi, jnp.float32)
  for _ in range(3):
    y = y * (1.5 - 0.5 * x * y * y)
  return y


def _sort16(v):
  return lax.sort(v, dimension=0, num_keys=1)


def _merge16(a, b):
  return _sort16(jnp.maximum(a, lax.rev(b, (0,))))


def _top16(vals):
  t = [_sort16(v) for v in vals]
  while len(t) > 1:
    t = [_merge16(t[2 * i], t[2 * i + 1]) for i in range(len(t) // 2)]
  return t[0]


def _phase_a_body(table, ida, idb, alphav, u_out,
                  ida_v, idb_v, al_v, ea, eb, ua, ub, zb,
                  sem_a, sem_b, sem_sa, sem_sb):
  cid = lax.axis_index("c")
  sid = lax.axis_index("s")
  wid = sid * NCORE + cid
  base = wid * PPW
  cpa = pltpu.async_copy(ida.at[pl.ds(base, PPW)], ida_v, sem_a)
  cpb = pltpu.async_copy(idb.at[pl.ds(base, PPW)], idb_v, sem_b)
  pltpu.sync_copy(alphav, al_v)
  alpha16 = al_v[...]

  z = jnp.zeros((LANES,), jnp.float32)
  for i in range(CHB):
    for j in range(NREG):
      zb[i, pl.ds(j * LANES, LANES)] = z

  @pl.when(wid == 0)
  def _():
    pltpu.sync_copy(zb, u_out.at[pl.ds(UZERO, CHB)])

  cpa.wait()
  cpb.wait()

  iot = lax.iota(jnp.int32, LANES)
  topmask = iot >= (LANES - KTOP)
  inf16 = jnp.full((LANES,), jnp.inf, jnp.float32)

  prev_stores = []
  for ci in range(PPW // CHA):
    ga = pltpu.async_copy(table.at[ida_v.at[pl.ds(ci * CHA, CHA)]], ea, sem_a)
    gb = pltpu.async_copy(table.at[idb_v.at[pl.ds(ci * CHA, CHA)]], eb, sem_b)
    ga.wait()
    gb.wait()
    for d in prev_stores:
      d.wait()

    def pair_body(p, carry):
      a = [ea[p, pl.ds(j * LANES, LANES)] for j in range(NREG)]
      b = [eb[p, pl.ds(j * LANES, LANES)] for j in range(NREG)]
      al = [jnp.abs(a[j] * b[j]) for j in range(NREG)]
      d = [a[j] - b[j] for j in range(NREG)]
      ss = d[0] * d[0]
      for j in range(1, NREG):
        ss = ss + d[j] * d[j]
      tot = jnp.sum(ss)
      x16 = jnp.full((LANES,), tot, jnp.float32)
      y16 = _rsqrt16(x16)
      t16 = _top16(al)
      thr = jnp.min(jnp.where(topmask, t16, inf16))
      thr16 = jnp.full((LANES,), thr, jnp.float32)
      s16 = alpha16 * y16
      for j in range(NREG):
        u = jnp.where(al[j] >= thr16, s16 * d[j], 0.0)
        ua[p, pl.ds(j * LANES, LANES)] = u
        ub[p, pl.ds(j * LANES, LANES)] = -u
      return carry

    lax.fori_loop(0, CHA, pair_body, 0)
    sa = pltpu.async_copy(ua, u_out.at[pl.ds(base + ci * CHA, CHA)], sem_sa)
    sb = pltpu.async_copy(ub, u_out.at[pl.ds(NB + base + ci * CHA, CHA)],
                          sem_sb)
    prev_stores = [sa, sb]
  for d in prev_stores:
    d.wait()


def _phase_b_body(table, ida, idb, u_in, out,
                  ida_v, idb_v, rows_l, items_l, lrows_l, zb, ubuf, tbuf,
                  sem1, sem2, sh_delta):
  cid = lax.axis_index("c")
  sid = lax.axis_index("s")
  tbase = sid * PPT
  cpa = pltpu.async_copy(ida.at[pl.ds(tbase, PPT)], ida_v, sem1)
  cpb = pltpu.async_copy(idb.at[pl.ds(tbase, PPT)], idb_v, sem2)

  z = jnp.zeros((LANES,), jnp.float32)

  def zb_body(i, carry):
    for j in range(NREG):
      zb[i, pl.ds(j * LANES, LANES)] = z
    return carry
  lax.fori_loop(0, CHL, zb_body, 0)

  cpa.wait()
  cpb.wait()
  iot = lax.iota(jnp.int32, LANES)

  for sl in range(NSLICE // NCORE):
    s = sl * NCORE + cid
    lo = s * SLICE_R
    lo16 = jnp.full((LANES,), lo, jnp.int32)
    uz16 = jnp.full((LANES,), UZERO, jnp.int32)

    def fill_body(q, carry):
      for j in range(NREG):
        rows_l[q, pl.ds(j * LANES, LANES)] = lo16
        items_l[q, pl.ds(j * LANES, LANES)] = uz16
      return carry
    lax.fori_loop(0, MAXCH, fill_body, 0)

    def make_scan(idv, item_off):
      def ch_body(ch, cnt):
        r = idv[pl.ds(ch * LANES, LANES)]
        m = (r >= lo16) & (r < lo16 + SLICE_R)
        pos = cnt + plsc.cumsum(m.astype(jnp.int32)) - 1
        i0 = lax.shift_right_logical(pos, 7)
        i1 = pos & (CHL - 1)
        plsc.store_scatter(rows_l, [i0, i1], r, mask=m)
        it = item_off + ch * LANES + iot
        plsc.store_scatter(items_l, [i0, i1], it, mask=m)
        return cnt + plsc.all_reduce_population_count(m)
      return ch_body

    cnt = jnp.zeros((LANES,), jnp.int32)
    cnt = lax.fori_loop(0, PPT // LANES, make_scan(ida_v, tbase), cnt)
    cnt = lax.fori_loop(0, PPT // LANES, make_scan(idb_v, NB + tbase), cnt)
    msc = jnp.max(cnt)
    nfull = lax.shift_right_logical(msc, 7)
    rem = msc & (CHL - 1)
    nrem = lax.shift_right_logical(rem + (LANES - 1), 4)

    def lr_body(q, carry):
      for j in range(NREG):
        lrows_l[q, pl.ds(j * LANES, LANES)] = (
            rows_l[q, pl.ds(j * LANES, LANES)] - lo16)
      return carry
    lax.fori_loop(0, MAXCH, lr_body, 0)

    @pl.when(sid == 0)
    def _():
      pltpu.sync_copy(zb.at[pl.ds(0, CHB)], sh_delta.at[pl.ds(0, CHB)])

    def zch(i, carry):
      pltpu.sync_copy(zb, sh_delta.at[lrows_l.at[i]])
      return carry
    lax.fori_loop(0, nfull, zch, 0)

    def zch16(k, carry):
      lrow = lrows_l[nfull, pl.ds(k * LANES, LANES)]
      pltpu.sync_copy(zb.at[pl.ds(0, LANES)], sh_delta.at[lrow])
      return carry
    lax.fori_loop(0, nrem, zch16, 0)
    plsc.subcore_barrier()

    def ach(i, carry):
      pltpu.sync_copy(u_in.at[items_l.at[i]], ubuf)
      pltpu.sync_copy(ubuf, sh_delta.at[lrows_l.at[i]], add=True)
      return carry
    lax.fori_loop(0, nfull, ach, 0)

    def ach16(k, carry):
      itv = items_l[nfull, pl.ds(k * LANES, LANES)]
      lrow = lrows_l[nfull, pl.ds(k * LANES, LANES)]
      pltpu.sync_copy(u_in.at[itv], ubuf.at[pl.ds(0, LANES)])
      pltpu.sync_copy(ubuf.at[pl.ds(0, LANES)], sh_delta.at[lrow], add=True)
      return carry
    lax.fori_loop(0, nrem, ach16, 0)
    plsc.subcore_barrier()

    def _apply(rows_src, lrows_src, nrows):
      c1 = pltpu.async_copy(table.at[rows_src], tbuf.at[pl.ds(0, nrows)],
                            sem1)
      c2 = pltpu.async_copy(sh_delta.at[lrows_src],
                            ubuf.at[pl.ds(0, nrows)], sem2)
      c1.wait()
      c2.wait()

      def add_body(rr, cc):
        for j in range(NREG):
          tbuf[rr, pl.ds(j * LANES, LANES)] = (
              tbuf[rr, pl.ds(j * LANES, LANES)]
              + ubuf[rr, pl.ds(j * LANES, LANES)])
        return cc
      lax.fori_loop(0, nrows, add_body, 0)
      pltpu.sync_copy(tbuf.at[pl.ds(0, nrows)], out.at[rows_src])

    def pch(i, carry):
      _apply(rows_l.at[i], lrows_l.at[i], CHL)
      return carry
    lax.fori_loop(0, nfull, pch, 0)

    def pch16(k, carry):
      grow = rows_l[nfull, pl.ds(k * LANES, LANES)]
      lrow = lrows_l[nfull, pl.ds(k * LANES, LANES)]
      _apply(grow, lrow, LANES)
      return carry
    lax.fori_loop(0, nrem, pch16, 0)
    plsc.subcore_barrier()


def _make_kernels():
  mesh = plsc.VectorSubcoreMesh(core_axis_name="c", subcore_axis_name="s",
                                num_cores=NCORE, num_subcores=NSUB)
  params = pltpu.CompilerParams(needs_layout_passes=False)
  phase_a = pl.kernel(
      _phase_a_body,
      out_type=jax.ShapeDtypeStruct((2 * NB + CHB, DIM), jnp.float32),
      mesh=mesh,
      compiler_params=params,
      scratch_types=[
          pltpu.VMEM((PPW,), jnp.int32),
          pltpu.VMEM((PPW,), jnp.int32),
          pltpu.VMEM((LANES,), jnp.float32),
          pltpu.VMEM((CHA, DIM), jnp.float32),
          pltpu.VMEM((CHA, DIM), jnp.float32),
          pltpu.VMEM((CHA, DIM), jnp.float32),
          pltpu.VMEM((CHA, DIM), jnp.float32),
          pltpu.VMEM((CHB, DIM), jnp.float32),
          pltpu.SemaphoreType.DMA,
          pltpu.SemaphoreType.DMA,
          pltpu.SemaphoreType.DMA,
          pltpu.SemaphoreType.DMA,
      ],
  )
  phase_b = pl.kernel(
      _phase_b_body,
      out_type=(),
      mesh=mesh,
      compiler_params=params,
      scratch_types=[
          pltpu.VMEM((PPT,), jnp.int32),
          pltpu.VMEM((PPT,), jnp.int32),
          pltpu.VMEM((MAXCH, CHL), jnp.int32),
          pltpu.VMEM((MAXCH, CHL), jnp.int32),
          pltpu.VMEM((MAXCH, CHL), jnp.int32),
          pltpu.VMEM((CHL, DIM), jnp.float32),
          pltpu.VMEM((CHL, DIM), jnp.float32),
          pltpu.VMEM((CHL, DIM), jnp.float32),
          pltpu.SemaphoreType.DMA,
          pltpu.SemaphoreType.DMA,
          pltpu.VMEM_SHARED((SLICE_R, DIM), jnp.float32),
      ],
  )
  return phase_a, phase_b


_PHASE_A, _PHASE_B = None, None


def kernel(table, token_ids_a, token_ids_b, alpha):
  global _PHASE_A, _PHASE_B
  if _PHASE_A is None:
    _PHASE_A, _PHASE_B = _make_kernels()
  alpha16 = jnp.broadcast_to(alpha.astype(jnp.float32), (LANES,))
  u = _PHASE_A(table, token_ids_a, token_ids_b, alpha16)
  out_ref = jax.new_ref(table)
  _PHASE_B(table, token_ids_a, token_ids_b, u, out_ref)
  return jax.freeze(out_ref)


# speedup vs baseline: 6.1861x; 1.0909x over previous
import jax
import jax.numpy as jnp
from jax import lax
from jax.experimental import pallas as pl
from jax.experimental.pallas import tpu as pltpu
from jax.experimental.pallas import tpu_sc as plsc

VOCAB = 100000
DIM = 128
NB = 16384
KTOP = 8
LANES = 16
NCORE = 2
NSUB = 16
NW = NCORE * NSUB
PPW = NB // NW
CHA = 128
NREG = DIM // LANES
NSLICE = 10
SLICE_R = VOCAB // NSLICE
RPT = SLICE_R // NSUB
ZROWS = 32
PPT = NB // NSUB
MAXM = 2 * PPT
CHB = 16
CHL = 128
MAXCH = MAXM // CHL + 1
UZERO = 2 * NB


def _rsqrt16(x):
  xi = plsc.bitcast(x, jnp.int32)
  yi = jnp.int32(0x5F3759DF) - lax.shift_right_logical(xi, 1)
  y = plsc.bitcast(yi, jnp.float32)
  for _ in range(3):
    y = y * (1.5 - 0.5 * x * y * y)
  return y


def _sort16(v):
  return lax.sort(v, dimension=0, num_keys=1)


def _merge16(a, b):
  return _sort16(jnp.maximum(a, lax.rev(b, (0,))))


def _top16(vals):
  t = [_sort16(v) for v in vals]
  while len(t) > 1:
    t = [_merge16(t[2 * i], t[2 * i + 1]) for i in range(len(t) // 2)]
  return t[0]


def _phase_a_body(table, ida, idb, alphav, u_out,
                  ida_v, idb_v, al_v, ea, eb, ua, ub, zb,
                  sem_a, sem_b, sem_sa, sem_sb):
  cid = lax.axis_index("c")
  sid = lax.axis_index("s")
  wid = sid * NCORE + cid
  base = wid * PPW
  cpa = pltpu.async_copy(ida.at[pl.ds(base, PPW)], ida_v, sem_a)
  cpb = pltpu.async_copy(idb.at[pl.ds(base, PPW)], idb_v, sem_b)
  pltpu.sync_copy(alphav, al_v)
  alpha16 = al_v[...]

  z = jnp.zeros((LANES,), jnp.float32)
  for i in range(CHB):
    for j in range(NREG):
      zb[i, pl.ds(j * LANES, LANES)] = z

  @pl.when(wid == 0)
  def _():
    pltpu.sync_copy(zb, u_out.at[pl.ds(UZERO, CHB)])

  cpa.wait()
  cpb.wait()

  iot = lax.iota(jnp.int32, LANES)
  topmask = iot >= (LANES - KTOP)
  inf16 = jnp.full((LANES,), jnp.inf, jnp.float32)

  prev_stores = []
  for ci in range(PPW // CHA):
    ga = pltpu.async_copy(table.at[ida_v.at[pl.ds(ci * CHA, CHA)]], ea, sem_a)
    gb = pltpu.async_copy(table.at[idb_v.at[pl.ds(ci * CHA, CHA)]], eb, sem_b)
    ga.wait()
    gb.wait()
    for d in prev_stores:
      d.wait()

    def pair_body(p, carry):
      a = [ea[p, pl.ds(j * LANES, LANES)] for j in range(NREG)]
      b = [eb[p, pl.ds(j * LANES, LANES)] for j in range(NREG)]
      al = [jnp.abs(a[j] * b[j]) for j in range(NREG)]
      d = [a[j] - b[j] for j in range(NREG)]
      ss = d[0] * d[0]
      for j in range(1, NREG):
        ss = ss + d[j] * d[j]
      tot = jnp.sum(ss)
      x16 = jnp.full((LANES,), tot, jnp.float32)
      y16 = _rsqrt16(x16)
      t16 = _top16(al)
      thr = jnp.min(jnp.where(topmask, t16, inf16))
      thr16 = jnp.full((LANES,), thr, jnp.float32)
      s16 = alpha16 * y16
      for j in range(NREG):
        u = jnp.where(al[j] >= thr16, s16 * d[j], 0.0)
        ua[p, pl.ds(j * LANES, LANES)] = u
        ub[p, pl.ds(j * LANES, LANES)] = -u
      return carry

    lax.fori_loop(0, CHA, pair_body, 0)
    sa = pltpu.async_copy(ua, u_out.at[pl.ds(base + ci * CHA, CHA)], sem_sa)
    sb = pltpu.async_copy(ub, u_out.at[pl.ds(NB + base + ci * CHA, CHA)],
                          sem_sb)
    prev_stores = [sa, sb]
  for d in prev_stores:
    d.wait()


def _phase_b_body(table, ida, idb, u_in, out,
                  ida_v, idb_v, items_l, lrows_l, zb, ubuf, tbuf,
                  s_ta, s_tb, s_da, s_db, s_wa, s_wb, s_z, sh_delta):
  cid = lax.axis_index("c")
  sid = lax.axis_index("s")
  tbase = sid * PPT
  cpa = pltpu.async_copy(ida.at[pl.ds(tbase, PPT)], ida_v, s_ta)
  cpb = pltpu.async_copy(idb.at[pl.ds(tbase, PPT)], idb_v, s_da)

  z = jnp.zeros((LANES,), jnp.float32)

  def zb_body(i, carry):
    for j in range(NREG):
      zb[i, pl.ds(j * LANES, LANES)] = z
    return carry
  lax.fori_loop(0, ZROWS, zb_body, 0)

  cpa.wait()
  cpb.wait()
  iot = lax.iota(jnp.int32, LANES)

  dense_chunks = [(i * 64, 64) for i in range(RPT // 64)]
  if RPT % 64:
    dense_chunks.append((RPT - RPT % 64, RPT % 64))
  zero_chunks = [(i * ZROWS, ZROWS) for i in range(RPT // ZROWS)]
  if RPT % ZROWS:
    zero_chunks.append((RPT - RPT % ZROWS, RPT % ZROWS))

  for sl in range(NSLICE // NCORE):
    s = sl * NCORE + cid
    lo = s * SLICE_R
    lo16 = jnp.full((LANES,), lo, jnp.int32)
    uz16 = jnp.full((LANES,), UZERO, jnp.int32)
    dbase = sid * RPT

    zdescs = []
    for off, szc in zero_chunks:
      zdescs.append(pltpu.async_copy(
          zb.at[pl.ds(0, szc)], sh_delta.at[pl.ds(dbase + off, szc)], s_z))

    def fill_body(q, carry):
      for j in range(NREG):
        lrows_l[q, pl.ds(j * LANES, LANES)] = jnp.zeros((LANES,), jnp.int32)
        items_l[q, pl.ds(j * LANES, LANES)] = uz16
      return carry
    lax.fori_loop(0, MAXCH, fill_body, 0)

    def make_scan(idv, item_off):
      def ch_body(ch, cnt):
        r = idv[pl.ds(ch * LANES, LANES)]
        m = (r >= lo16) & (r < lo16 + SLICE_R)
        pos = cnt + plsc.cumsum(m.astype(jnp.int32)) - 1
        i0 = lax.shift_right_logical(pos, 7)
        i1 = pos & (CHL - 1)
        plsc.store_scatter(lrows_l, [i0, i1], r - lo16, mask=m)
        it = item_off + ch * LANES + iot
        plsc.store_scatter(items_l, [i0, i1], it, mask=m)
        return cnt + plsc.all_reduce_population_count(m)
      return ch_body

    cnt = jnp.zeros((LANES,), jnp.int32)
    cnt = lax.fori_loop(0, PPT // LANES, make_scan(ida_v, tbase), cnt)
    cnt = lax.fori_loop(0, PPT // LANES, make_scan(idb_v, NB + tbase), cnt)
    msc = jnp.max(cnt)
    nfull = lax.shift_right_logical(msc, 7)
    rem = msc & (CHL - 1)
    nrem = lax.shift_right_logical(rem + (LANES - 1), 4)

    for dsc in zdescs:
      dsc.wait()
    plsc.subcore_barrier()

    def ach(i, carry):
      pltpu.sync_copy(u_in.at[items_l.at[i]], ubuf)
      pltpu.sync_copy(ubuf, sh_delta.at[lrows_l.at[i]], add=True)
      return carry
    lax.fori_loop(0, nfull, ach, 0)

    def ach16(k, carry):
      itv = items_l[nfull, pl.ds(k * LANES, LANES)]
      lrow = lrows_l[nfull, pl.ds(k * LANES, LANES)]
      pltpu.sync_copy(u_in.at[itv], ubuf.at[pl.ds(0, LANES)])
      pltpu.sync_copy(ubuf.at[pl.ds(0, LANES)], sh_delta.at[lrow], add=True)
      return carry
    lax.fori_loop(0, nrem, ach16, 0)
    plsc.subcore_barrier()

    gsems = [(s_ta, s_da), (s_tb, s_db)]
    wsems = [s_wa, s_wb]
    gd = [None, None]
    wd = [None, None]

    def issue(k):
      off, szc = dense_chunks[k]
      par = k % 2
      tv = tbuf.at[pl.ds(par * 64, szc)]
      uv = ubuf.at[pl.ds(par * 64, szc)]
      g1 = pltpu.async_copy(table.at[pl.ds(lo + dbase + off, szc)], tv,
                            gsems[par][0])
      g2 = pltpu.async_copy(sh_delta.at[pl.ds(dbase + off, szc)], uv,
                            gsems[par][1])
      gd[par] = (g1, g2)

    issue(0)
    for k in range(len(dense_chunks)):
      off, szc = dense_chunks[k]
      par = k % 2
      if k + 1 < len(dense_chunks):
        if wd[(k + 1) % 2] is not None:
          wd[(k + 1) % 2].wait()
          wd[(k + 1) % 2] = None
        issue(k + 1)
      g1, g2 = gd[par]
      g1.wait()
      g2.wait()

      def add_body(rr, cc):
        trr = par * 64 + rr
        for j in range(NREG):
          tbuf[trr, pl.ds(j * LANES, LANES)] = (
              tbuf[trr, pl.ds(j * LANES, LANES)]
              + ubuf[trr, pl.ds(j * LANES, LANES)])
        return cc
      lax.fori_loop(0, szc, add_body, 0)
      wd[par] = pltpu.async_copy(
          tbuf.at[pl.ds(par * 64, szc)],
          out.at[pl.ds(lo + dbase + off, szc)], wsems[par])
    for par in range(2):
      if wd[par] is not None:
        wd[par].wait()
    plsc.subcore_barrier()


def _make_kernels():
  mesh = plsc.VectorSubcoreMesh(core_axis_name="c", subcore_axis_name="s",
                                num_cores=NCORE, num_subcores=NSUB)
  params = pltpu.CompilerParams(needs_layout_passes=False,
                                use_tc_tiling_on_sc=False)
  phase_a = pl.kernel(
      _phase_a_body,
      out_type=jax.ShapeDtypeStruct((2 * NB + CHB, DIM), jnp.float32),
      mesh=mesh,
      compiler_params=params,
      scratch_types=[
          pltpu.VMEM((PPW,), jnp.int32),
          pltpu.VMEM((PPW,), jnp.int32),
          pltpu.VMEM((LANES,), jnp.float32),
          pltpu.VMEM((CHA, DIM), jnp.float32),
          pltpu.VMEM((CHA, DIM), jnp.float32),
          pltpu.VMEM((CHA, DIM), jnp.float32),
          pltpu.VMEM((CHA, DIM), jnp.float32),
          pltpu.VMEM((CHB, DIM), jnp.float32),
          pltpu.SemaphoreType.DMA,
          pltpu.SemaphoreType.DMA,
          pltpu.SemaphoreType.DMA,
          pltpu.SemaphoreType.DMA,
      ],
  )
  phase_b = pl.kernel(
      _phase_b_body,
      out_type=jax.ShapeDtypeStruct((VOCAB, DIM), jnp.float32),
      mesh=mesh,
      compiler_params=params,
      scratch_types=[
          pltpu.VMEM((PPT,), jnp.int32),
          pltpu.VMEM((PPT,), jnp.int32),
          pltpu.VMEM((MAXCH, CHL), jnp.int32),
          pltpu.VMEM((MAXCH, CHL), jnp.int32),
          pltpu.VMEM((ZROWS, DIM), jnp.float32),
          pltpu.VMEM((CHL, DIM), jnp.float32),
          pltpu.VMEM((CHL, DIM), jnp.float32),
          pltpu.SemaphoreType.DMA,
          pltpu.SemaphoreType.DMA,
          pltpu.SemaphoreType.DMA,
          pltpu.SemaphoreType.DMA,
          pltpu.SemaphoreType.DMA,
          pltpu.SemaphoreType.DMA,
          pltpu.SemaphoreType.DMA,
          pltpu.VMEM_SHARED((SLICE_R, DIM), jnp.float32),
      ],
  )
  return phase_a, phase_b


_PHASE_A, _PHASE_B = None, None


def kernel(table, token_ids_a, token_ids_b, alpha):
  global _PHASE_A, _PHASE_B
  if _PHASE_A is None:
    _PHASE_A, _PHASE_B = _make_kernels()
  alpha16 = jnp.broadcast_to(alpha.astype(jnp.float32), (LANES,))
  u = _PHASE_A(table, token_ids_a, token_ids_b, alpha16)
  return _PHASE_B(table, token_ids_a, token_ids_b, u)
